# baseline (speedup 1.0000x reference)
import functools

import jax
import jax.numpy as jnp
from jax import lax
from jax.experimental import pallas as pl
from jax.experimental.pallas import tpu as pltpu
from jax.experimental.pallas import tpu_sc as plsc

F32 = jnp.float32
HALF = 64
CW = 8
TE = 256
GSZ = 8
NBUF = 2
NS = 16


@functools.lru_cache(None)
def _make_agg(n_acc, n_out, e_rows):
    rows_per_tile = e_rows // NS
    zrows = n_acc // NS // TE
    zrem = n_acc // NS - zrows * TE
    orows = n_out // NS
    n_groups = rows_per_tile // GSZ
    tail = rows_per_tile - n_groups * GSZ

    def body(tabL, tabR, src2d, dst2d, zeros_in, zeros8_in, ones_in,
             sumL, sumR, cnt_out,
             acc, cnt_acc, src_v, dst_v,
             rows0, rows1, ones_v,
             gsem0, gsem1, ssem0, ssem1, csem0, csem1):
        cid = lax.axis_index("c")
        sid = lax.axis_index("s")
        pltpu.sync_copy(zeros_in, rows0)
        pltpu.sync_copy(zeros8_in, ones_v)
        rows = (rows0, rows1)
        gsem = (gsem0, gsem1)
        ssem = (ssem0, ssem1)
        csem = (csem0, csem1)

        zbase = sid * (n_acc // NS)

        for i in range(zrows):
            pltpu.async_copy(rows0, acc.at[pl.ds(zbase + i * TE, TE), :],
                             gsem[i % 2])
            pltpu.async_copy(ones_v,
                             cnt_acc.at[pl.ds(zbase + i * TE, TE), :],
                             csem[i % 2])
        if zrem:
            zoff = zbase + zrows * TE
            pltpu.async_copy(rows0.at[0:zrem, :],
                             acc.at[pl.ds(zoff, zrem), :], ssem0)
            pltpu.async_copy(ones_v.at[0:zrem, :],
                             cnt_acc.at[pl.ds(zoff, zrem), :], ssem1)
        for i in range(zrows):
            pltpu.make_async_copy(rows0, acc.at[pl.ds(zbase + i * TE, TE), :],
                                  gsem[i % 2]).wait()
            pltpu.make_async_copy(ones_v,
                                  cnt_acc.at[pl.ds(zbase + i * TE, TE), :],
                                  csem[i % 2]).wait()
        if zrem:
            zoff = zbase + zrows * TE
            pltpu.make_async_copy(rows0.at[0:zrem, :],
                                  acc.at[pl.ds(zoff, zrem), :], ssem0).wait()
            pltpu.make_async_copy(ones_v.at[0:zrem, :],
                                  cnt_acc.at[pl.ds(zoff, zrem), :],
                                  ssem1).wait()
        pltpu.sync_copy(ones_in, ones_v)
        plsc.subcore_barrier()

        rbase = sid * rows_per_tile

        def do_group(tab, with_cnt, base, n):
            pltpu.sync_copy(src2d.at[pl.ds(base, n)], src_v.at[0:n, :])
            pltpu.sync_copy(dst2d.at[pl.ds(base, n)], dst_v.at[0:n, :])

            m = n

            def issue_gather(k):
                pltpu.async_copy(tab.at[src_v.at[k]],
                                 rows[k % NBUF], gsem[k % NBUF])

            def wait_gather(k):
                pltpu.make_async_copy(tab.at[src_v.at[k]],
                                      rows[k % NBUF], gsem[k % NBUF]).wait()

            def issue_scatter(k):
                pltpu.async_copy(rows[k % NBUF],
                                 acc.at[dst_v.at[k]],
                                 ssem[k % NBUF], add=True)

            def wait_scatter(k):
                pltpu.make_async_copy(rows[k % NBUF],
                                      acc.at[dst_v.at[k]],
                                      ssem[k % NBUF]).wait()

            def issue_cnt(k):
                pltpu.async_copy(ones_v,
                                 cnt_acc.at[dst_v.at[k]],
                                 csem[k % 2], add=True)

            def wait_cnt(k):
                pltpu.make_async_copy(ones_v,
                                      cnt_acc.at[dst_v.at[k]],
                                      csem[k % 2]).wait()

            for k in range(min(NBUF - 1, m)):
                issue_gather(k)
            for j in range(m):
                k = j + NBUF - 1
                if k < m:
                    if k - NBUF >= 0:
                        wait_scatter(k - NBUF)
                    issue_gather(k)
                wait_gather(j)
                if with_cnt:
                    if j >= 2:
                        wait_cnt(j - 2)
                    issue_cnt(j)
                issue_scatter(j)
            for j in range(max(0, m - NBUF), m):
                wait_scatter(j)
            if with_cnt:
                for j in range(max(0, m - 2), m):
                    wait_cnt(j)

        def make_grp(tab, with_cnt):
            def grp(g, carry):
                do_group(tab, with_cnt, rbase + g * GSZ, GSZ)
                return carry
            return grp

        def run_core(tab, with_cnt):
            lax.fori_loop(0, n_groups, make_grp(tab, with_cnt), 0)
            if tail:
                do_group(tab, with_cnt, rbase + n_groups * GSZ, tail)

        @pl.when(cid == 0)
        def _():
            run_core(tabL, True)

        @pl.when(cid == 1)
        def _():
            run_core(tabR, False)

        plsc.subcore_barrier()

        ob = sid * orows

        @pl.when(cid == 0)
        def _():
            pltpu.sync_copy(acc.at[pl.ds(ob, orows), :],
                            sumL.at[pl.ds(ob, orows), :])
            pltpu.sync_copy(cnt_acc.at[pl.ds(ob, orows), :],
                            cnt_out.at[pl.ds(ob, orows), :])

        @pl.when(cid == 1)
        def _():
            pltpu.sync_copy(acc.at[pl.ds(ob, orows), :],
                            sumR.at[pl.ds(ob, orows), :])

    mesh = plsc.VectorSubcoreMesh(core_axis_name="c", subcore_axis_name="s")
    return pl.kernel(
        body,
        out_type=(
            jax.ShapeDtypeStruct((n_out, HALF), F32),
            jax.ShapeDtypeStruct((n_out, HALF), F32),
            jax.ShapeDtypeStruct((n_out, CW), F32),
        ),
        mesh=mesh,
        compiler_params=pltpu.CompilerParams(use_tc_tiling_on_sc=False),
        scratch_types=[
            pltpu.VMEM_SHARED((n_acc, HALF), F32),
            pltpu.VMEM_SHARED((n_acc, CW), F32),
            pltpu.VMEM((GSZ, TE), jnp.int32),
            pltpu.VMEM((GSZ, TE), jnp.int32),
            pltpu.VMEM((TE, HALF), F32),
            pltpu.VMEM((TE, HALF), F32),
            pltpu.VMEM((TE, CW), F32),
        ] + [pltpu.SemaphoreType.DMA] * 6,
    )


def _dense_call(xd, sL, sR, cnt, Wself, WnL, WnR, b, mode,
                Wlin=None, blin=None, blk=512):
    n = xd.shape[0]

    def kb(x_ref, sl_ref, sr_ref, c_ref, ws_ref, wl_ref, wr_ref, b_ref, *rest):
        inv = 1.0 / jnp.maximum(
            jnp.max(c_ref[...], axis=1, keepdims=True), 1.0)
        h = jnp.dot(x_ref[...], ws_ref[...], preferred_element_type=F32)
        h = h + jnp.dot(sl_ref[...] * inv, wl_ref[...],
                        preferred_element_type=F32)
        h = h + jnp.dot(sr_ref[...] * inv, wr_ref[...],
                        preferred_element_type=F32)
        h = jnp.maximum(h + b_ref[...], 0.0)
        if mode == 'final':
            wlin_ref, blin_ref, o_ref = rest
            o_ref[...] = jnp.dot(h, wlin_ref[...],
                                 preferred_element_type=F32) + blin_ref[...]
        elif mode == 'split':
            oL_ref, oR_ref = rest
            oL_ref[...] = h[:, :HALF]
            oR_ref[...] = h[:, HALF:]
        else:
            o_ref, = rest
            o_ref[...] = h

    full = lambda r, c: pl.BlockSpec((r, c), lambda i: (0, 0))
    rowb = lambda r, c: pl.BlockSpec((r, c), lambda i: (i, 0))
    in_specs = [rowb(blk, 128), rowb(blk, HALF), rowb(blk, HALF),
                rowb(blk, CW), full(128, 128), full(HALF, 128),
                full(HALF, 128), full(1, 128)]
    args = [xd, sL, sR, cnt, Wself, WnL, WnR, b.reshape(1, 128)]
    if mode == 'final':
        in_specs += [full(128, 128), full(1, 128)]
        args += [Wlin, blin.reshape(1, 128)]
        out_shape = jax.ShapeDtypeStruct((n, 128), F32)
        out_specs = rowb(blk, 128)
    elif mode == 'split':
        out_shape = (jax.ShapeDtypeStruct((n, HALF), F32),
                     jax.ShapeDtypeStruct((n, HALF), F32))
        out_specs = (rowb(blk, HALF), rowb(blk, HALF))
    else:
        out_shape = jax.ShapeDtypeStruct((n, 128), F32)
        out_specs = rowb(blk, 128)
    return pl.pallas_call(kb, grid=(n // blk,), in_specs=in_specs,
                          out_specs=out_specs, out_shape=out_shape)(*args)


def _prep_edges(src, dst, junk, e_pad):
    e = src.shape[0]
    src = jnp.concatenate([src, jnp.zeros((e_pad - e,), jnp.int32)])
    dst = jnp.concatenate([dst, jnp.full((e_pad - e,), junk, jnp.int32)])
    return src.reshape(-1, TE), dst.reshape(-1, TE)


def kernel(x_item, x_user, ii_src, ii_dst, iu0_src, iu0_dst, iu1_src, iu1_dst,
           W_self1, W_neigh1, b1, W_self2, W_neigh2, b2,
           W_self3, W_neigh3, b3, W_lin, b_lin):
    zeros = jnp.zeros((TE, HALF), F32)
    zeros8 = jnp.zeros((TE, CW), F32)
    ones = jnp.ones((TE, CW), F32)
    xiL = x_item[:, :HALF]
    xiR = x_item[:, HALF:]

    s1, d1 = _prep_edges(ii_src, ii_dst, 20000, 606208)
    sum1L, sum1R, cnt1 = _make_agg(20480, 20480, 606208 // TE)(
        xiL, xiR, s1, d1, zeros, zeros8, ones)
    itemL, itemR = _dense_call(x_item[:20480], sum1L, sum1R, cnt1,
                               W_self1, W_neigh1[:HALF], W_neigh1[HALF:],
                               b1, 'split')

    s2, d2 = _prep_edges(iu0_src, iu0_dst, 10000, 327680)
    sum2L, sum2R, cnt2 = _make_agg(10240, 5120, 327680 // TE)(
        xiL, xiR, s2, d2, zeros, zeros8, ones)
    user5 = _dense_call(x_user[:5120], sum2L, sum2R, cnt2,
                        W_self2, W_neigh2[:HALF], W_neigh2[HALF:],
                        b2, 'plain')

    s3, d3 = _prep_edges(iu1_src, iu1_dst, 5000, 163840)
    sum3L, sum3R, cnt3 = _make_agg(5120, 5120, 163840 // TE)(
        itemL, itemR, s3, d3, zeros, zeros8, ones)
    out = _dense_call(user5, sum3L, sum3R, cnt3,
                      W_self3, W_neigh3[:HALF], W_neigh3[HALF:], b3,
                      'final', W_lin, b_lin)
    return out[:5000]


# speedup vs baseline: 1.0383x; 1.0383x over previous
import functools

import jax
import jax.numpy as jnp
from jax import lax
from jax.experimental import pallas as pl
from jax.experimental.pallas import tpu as pltpu
from jax.experimental.pallas import tpu_sc as plsc

F32 = jnp.float32
HALF = 64
CW = 8
GSZ = 16
NBUF = 4
NS = 16


@functools.lru_cache(None)
def _make_agg(n_acc, n_out, e_rows):
    rows_per_tile = e_rows // NS
    zrows = n_acc // NS // 128
    zrem = n_acc // NS - zrows * 128
    orows = n_out // NS
    n_groups = rows_per_tile // GSZ
    tail = rows_per_tile - n_groups * GSZ

    def body(tabL, tabR, src2d, dst2d, zeros_in, zeros8_in, ones_in,
             sumL, sumR, cnt_out,
             acc, cnt_acc, src_v, dst_v,
             rows0, rows1, rows2, rows3, ones_v,
             gsem0, gsem1, gsem2, gsem3,
             ssem0, ssem1, ssem2, ssem3, csem0, csem1):
        cid = lax.axis_index("c")
        sid = lax.axis_index("s")
        pltpu.sync_copy(zeros_in, rows0)
        pltpu.sync_copy(zeros8_in, ones_v)
        rows = (rows0, rows1, rows2, rows3)
        gsem = (gsem0, gsem1, gsem2, gsem3)
        ssem = (ssem0, ssem1, ssem2, ssem3)
        csem = (csem0, csem1)

        zbase = sid * (n_acc // NS)

        def zero_body(i, carry):
            pltpu.sync_copy(rows0, acc.at[pl.ds(zbase + i * 128, 128), :])
            pltpu.sync_copy(ones_v,
                            cnt_acc.at[pl.ds(zbase + i * 128, 128), :])
            return carry

        lax.fori_loop(0, zrows, zero_body, 0)
        if zrem:
            zoff = zbase + zrows * 128
            pltpu.sync_copy(rows0.at[0:zrem, :], acc.at[pl.ds(zoff, zrem), :])
            pltpu.sync_copy(ones_v.at[0:zrem, :],
                            cnt_acc.at[pl.ds(zoff, zrem), :])
        pltpu.sync_copy(ones_in, ones_v)
        plsc.subcore_barrier()

        rbase = sid * rows_per_tile

        def do_group(tab, with_cnt, base, n):
            pltpu.sync_copy(src2d.at[pl.ds(base, n)], src_v.at[0:n, :])
            pltpu.sync_copy(dst2d.at[pl.ds(base, n)], dst_v.at[0:n, :])

            def issue_gather(k):
                pltpu.async_copy(tab.at[src_v.at[k]], rows[k % NBUF],
                                 gsem[k % NBUF])

            def wait_gather(k):
                pltpu.make_async_copy(tab.at[src_v.at[k]], rows[k % NBUF],
                                      gsem[k % NBUF]).wait()

            def issue_scatter(k):
                pltpu.async_copy(rows[k % NBUF], acc.at[dst_v.at[k]],
                                 ssem[k % NBUF], add=True)

            def wait_scatter(k):
                pltpu.make_async_copy(rows[k % NBUF], acc.at[dst_v.at[k]],
                                      ssem[k % NBUF]).wait()

            def issue_cnt(k):
                pltpu.async_copy(ones_v, cnt_acc.at[dst_v.at[k]],
                                 csem[k % 2], add=True)

            def wait_cnt(k):
                pltpu.make_async_copy(ones_v, cnt_acc.at[dst_v.at[k]],
                                      csem[k % 2]).wait()

            for k in range(min(NBUF - 1, n)):
                issue_gather(k)
            for j in range(n):
                k = j + NBUF - 1
                if k < n:
                    if k - NBUF >= 0:
                        wait_scatter(k - NBUF)
                    issue_gather(k)
                wait_gather(j)
                if with_cnt:
                    if j >= 2:
                        wait_cnt(j - 2)
                    issue_cnt(j)
                issue_scatter(j)
            for j in range(max(0, n - NBUF), n):
                wait_scatter(j)
            if with_cnt:
                for j in range(max(0, n - 2), n):
                    wait_cnt(j)

        def make_grp(tab, with_cnt):
            def grp(g, carry):
                do_group(tab, with_cnt, rbase + g * GSZ, GSZ)
                return carry
            return grp

        def run_core(tab, with_cnt):
            lax.fori_loop(0, n_groups, make_grp(tab, with_cnt), 0)
            if tail:
                do_group(tab, with_cnt, rbase + n_groups * GSZ, tail)

        @pl.when(cid == 0)
        def _():
            run_core(tabL, True)

        @pl.when(cid == 1)
        def _():
            run_core(tabR, False)

        plsc.subcore_barrier()

        ob = sid * orows

        @pl.when(cid == 0)
        def _():
            pltpu.sync_copy(acc.at[pl.ds(ob, orows), :],
                            sumL.at[pl.ds(ob, orows), :])
            pltpu.sync_copy(cnt_acc.at[pl.ds(ob, orows), :],
                            cnt_out.at[pl.ds(ob, orows), :])

        @pl.when(cid == 1)
        def _():
            pltpu.sync_copy(acc.at[pl.ds(ob, orows), :],
                            sumR.at[pl.ds(ob, orows), :])

    mesh = plsc.VectorSubcoreMesh(core_axis_name="c", subcore_axis_name="s")
    return pl.kernel(
        body,
        out_type=(
            jax.ShapeDtypeStruct((n_out, HALF), F32),
            jax.ShapeDtypeStruct((n_out, HALF), F32),
            jax.ShapeDtypeStruct((n_out, CW), F32),
        ),
        mesh=mesh,
        compiler_params=pltpu.CompilerParams(use_tc_tiling_on_sc=False),
        scratch_types=[
            pltpu.VMEM_SHARED((n_acc, HALF), F32),
            pltpu.VMEM_SHARED((n_acc, CW), F32),
            pltpu.VMEM((GSZ, 128), jnp.int32),
            pltpu.VMEM((GSZ, 128), jnp.int32),
            pltpu.VMEM((128, HALF), F32),
            pltpu.VMEM((128, HALF), F32),
            pltpu.VMEM((128, HALF), F32),
            pltpu.VMEM((128, HALF), F32),
            pltpu.VMEM((128, CW), F32),
        ] + [pltpu.SemaphoreType.DMA] * 10,
    )


def _make_agg23():
    n_acc = 15360
    ra, rb = 160, 80
    rpt = ra + rb
    zrows = n_acc // NS // 128
    orows = n_acc // NS
    zrem = orows - zrows * 128

    def body(tabAL, tabAR, tabBL, tabBR, src2d, dst2d,
             zeros_in, zeros8_in, ones_in,
             sumL, sumR, cnt_out,
             acc, cnt_acc, src_v, dst_v,
             rows0, rows1, rows2, rows3, ones_v,
             gsem0, gsem1, gsem2, gsem3,
             ssem0, ssem1, ssem2, ssem3, csem0, csem1):
        cid = lax.axis_index("c")
        sid = lax.axis_index("s")
        pltpu.sync_copy(zeros_in, rows0)
        pltpu.sync_copy(zeros8_in, ones_v)
        rows = (rows0, rows1, rows2, rows3)
        gsem = (gsem0, gsem1, gsem2, gsem3)
        ssem = (ssem0, ssem1, ssem2, ssem3)
        csem = (csem0, csem1)

        zbase = sid * orows

        def zero_body(i, carry):
            pltpu.sync_copy(rows0, acc.at[pl.ds(zbase + i * 128, 128), :])
            pltpu.sync_copy(ones_v,
                            cnt_acc.at[pl.ds(zbase + i * 128, 128), :])
            return carry

        lax.fori_loop(0, zrows, zero_body, 0)
        if zrem:
            zoff = zbase + zrows * 128
            pltpu.sync_copy(rows0.at[0:zrem, :],
                            acc.at[pl.ds(zoff, zrem), :])
            pltpu.sync_copy(ones_v.at[0:zrem, :],
                            cnt_acc.at[pl.ds(zoff, zrem), :])
        pltpu.sync_copy(ones_in, ones_v)
        plsc.subcore_barrier()

        rbase = sid * rpt

        def do_group(tab, with_cnt, base, n):
            pltpu.sync_copy(src2d.at[pl.ds(base, n)], src_v.at[0:n, :])
            pltpu.sync_copy(dst2d.at[pl.ds(base, n)], dst_v.at[0:n, :])

            def issue_gather(k):
                pltpu.async_copy(tab.at[src_v.at[k]], rows[k % NBUF],
                                 gsem[k % NBUF])

            def wait_gather(k):
                pltpu.make_async_copy(tab.at[src_v.at[k]], rows[k % NBUF],
                                      gsem[k % NBUF]).wait()

            def issue_scatter(k):
                pltpu.async_copy(rows[k % NBUF], acc.at[dst_v.at[k]],
                                 ssem[k % NBUF], add=True)

            def wait_scatter(k):
                pltpu.make_async_copy(rows[k % NBUF], acc.at[dst_v.at[k]],
                                      ssem[k % NBUF]).wait()

            def issue_cnt(k):
                pltpu.async_copy(ones_v, cnt_acc.at[dst_v.at[k]],
                                 csem[k % 2], add=True)

            def wait_cnt(k):
                pltpu.make_async_copy(ones_v, cnt_acc.at[dst_v.at[k]],
                                      csem[k % 2]).wait()

            for k in range(min(NBUF - 1, n)):
                issue_gather(k)
            for j in range(n):
                k = j + NBUF - 1
                if k < n:
                    if k - NBUF >= 0:
                        wait_scatter(k - NBUF)
                    issue_gather(k)
                wait_gather(j)
                if with_cnt:
                    if j >= 2:
                        wait_cnt(j - 2)
                    issue_cnt(j)
                issue_scatter(j)
            for j in range(max(0, n - NBUF), n):
                wait_scatter(j)
            if with_cnt:
                for j in range(max(0, n - 2), n):
                    wait_cnt(j)

        def run_seg(tab, with_cnt, seg_base, seg_rows):
            def grp(g, carry):
                do_group(tab, with_cnt, seg_base + g * GSZ, GSZ)
                return carry
            lax.fori_loop(0, seg_rows // GSZ, grp, 0)

        @pl.when(cid == 0)
        def _():
            run_seg(tabAL, True, rbase, ra)
            run_seg(tabBL, True, rbase + ra, rb)

        @pl.when(cid == 1)
        def _():
            run_seg(tabAR, False, rbase, ra)
            run_seg(tabBR, False, rbase + ra, rb)

        plsc.subcore_barrier()

        ob = sid * orows

        @pl.when(cid == 0)
        def _():
            pltpu.sync_copy(acc.at[pl.ds(ob, orows), :],
                            sumL.at[pl.ds(ob, orows), :])
            pltpu.sync_copy(cnt_acc.at[pl.ds(ob, orows), :],
                            cnt_out.at[pl.ds(ob, orows), :])

        @pl.when(cid == 1)
        def _():
            pltpu.sync_copy(acc.at[pl.ds(ob, orows), :],
                            sumR.at[pl.ds(ob, orows), :])

    mesh = plsc.VectorSubcoreMesh(core_axis_name="c", subcore_axis_name="s")
    return pl.kernel(
        body,
        out_type=(
            jax.ShapeDtypeStruct((n_acc, HALF), F32),
            jax.ShapeDtypeStruct((n_acc, HALF), F32),
            jax.ShapeDtypeStruct((n_acc, CW), F32),
        ),
        mesh=mesh,
        compiler_params=pltpu.CompilerParams(use_tc_tiling_on_sc=False),
        scratch_types=[
            pltpu.VMEM_SHARED((n_acc, HALF), F32),
            pltpu.VMEM_SHARED((n_acc, CW), F32),
            pltpu.VMEM((GSZ, 128), jnp.int32),
            pltpu.VMEM((GSZ, 128), jnp.int32),
            pltpu.VMEM((128, HALF), F32),
            pltpu.VMEM((128, HALF), F32),
            pltpu.VMEM((128, HALF), F32),
            pltpu.VMEM((128, HALF), F32),
            pltpu.VMEM((128, CW), F32),
        ] + [pltpu.SemaphoreType.DMA] * 10,
    )


def _dense23_call(xu, s2L, s2R, c2, s3L, s3R, c3,
                  W2s, W2nL, W2nR, b2, W3s, W3nL, W3nR, b3,
                  Wlin, blin, blk=512):
    n = xu.shape[0]

    def kb(xu_ref, s2l_ref, s2r_ref, c2_ref, s3l_ref, s3r_ref, c3_ref,
           w2s_ref, w2l_ref, w2r_ref, b2_ref,
           w3s_ref, w3l_ref, w3r_ref, b3_ref,
           wlin_ref, blin_ref, o_ref):
        inv2 = 1.0 / jnp.maximum(
            jnp.max(c2_ref[...], axis=1, keepdims=True), 1.0)
        u5 = jnp.dot(xu_ref[...], w2s_ref[...], preferred_element_type=F32)
        u5 = u5 + jnp.dot(s2l_ref[...] * inv2, w2l_ref[...],
                          preferred_element_type=F32)
        u5 = u5 + jnp.dot(s2r_ref[...] * inv2, w2r_ref[...],
                          preferred_element_type=F32)
        u5 = jnp.maximum(u5 + b2_ref[...], 0.0)
        inv3 = 1.0 / jnp.maximum(
            jnp.max(c3_ref[...], axis=1, keepdims=True), 1.0)
        h = jnp.dot(u5, w3s_ref[...], preferred_element_type=F32)
        h = h + jnp.dot(s3l_ref[...] * inv3, w3l_ref[...],
                        preferred_element_type=F32)
        h = h + jnp.dot(s3r_ref[...] * inv3, w3r_ref[...],
                        preferred_element_type=F32)
        h = jnp.maximum(h + b3_ref[...], 0.0)
        o_ref[...] = jnp.dot(h, wlin_ref[...],
                             preferred_element_type=F32) + blin_ref[...]

    full = lambda r, c: pl.BlockSpec((r, c), lambda i: (0, 0))
    rowb = lambda r, c: pl.BlockSpec((r, c), lambda i: (i, 0))
    in_specs = [rowb(blk, 128), rowb(blk, HALF), rowb(blk, HALF),
                rowb(blk, CW), rowb(blk, HALF), rowb(blk, HALF),
                rowb(blk, CW),
                full(128, 128), full(HALF, 128), full(HALF, 128),
                full(1, 128),
                full(128, 128), full(HALF, 128), full(HALF, 128),
                full(1, 128),
                full(128, 128), full(1, 128)]
    args = [xu, s2L, s2R, c2, s3L, s3R, c3,
            W2s, W2nL, W2nR, b2.reshape(1, 128),
            W3s, W3nL, W3nR, b3.reshape(1, 128),
            Wlin, blin.reshape(1, 128)]
    return pl.pallas_call(
        kb, grid=(n // blk,), in_specs=in_specs,
        out_specs=rowb(blk, 128),
        out_shape=jax.ShapeDtypeStruct((n, 128), F32))(*args)


def _dense_call(xd, sL, sR, cnt, Wself, WnL, WnR, b, mode,
                Wlin=None, blin=None, blk=512):
    n = xd.shape[0]

    def kb(x_ref, sl_ref, sr_ref, c_ref, ws_ref, wl_ref, wr_ref, b_ref, *rest):
        inv = 1.0 / jnp.maximum(
            jnp.max(c_ref[...], axis=1, keepdims=True), 1.0)
        h = jnp.dot(x_ref[...], ws_ref[...], preferred_element_type=F32)
        h = h + jnp.dot(sl_ref[...] * inv, wl_ref[...],
                        preferred_element_type=F32)
        h = h + jnp.dot(sr_ref[...] * inv, wr_ref[...],
                        preferred_element_type=F32)
        h = jnp.maximum(h + b_ref[...], 0.0)
        if mode == 'final':
            wlin_ref, blin_ref, o_ref = rest
            o_ref[...] = jnp.dot(h, wlin_ref[...],
                                 preferred_element_type=F32) + blin_ref[...]
        elif mode == 'split':
            oL_ref, oR_ref = rest
            oL_ref[...] = h[:, :HALF]
            oR_ref[...] = h[:, HALF:]
        else:
            o_ref, = rest
            o_ref[...] = h

    full = lambda r, c: pl.BlockSpec((r, c), lambda i: (0, 0))
    rowb = lambda r, c: pl.BlockSpec((r, c), lambda i: (i, 0))
    in_specs = [rowb(blk, 128), rowb(blk, HALF), rowb(blk, HALF),
                rowb(blk, CW), full(128, 128), full(HALF, 128),
                full(HALF, 128), full(1, 128)]
    args = [xd, sL, sR, cnt, Wself, WnL, WnR, b.reshape(1, 128)]
    if mode == 'final':
        in_specs += [full(128, 128), full(1, 128)]
        args += [Wlin, blin.reshape(1, 128)]
        out_shape = jax.ShapeDtypeStruct((n, 128), F32)
        out_specs = rowb(blk, 128)
    elif mode == 'split':
        out_shape = (jax.ShapeDtypeStruct((n, HALF), F32),
                     jax.ShapeDtypeStruct((n, HALF), F32))
        out_specs = (rowb(blk, HALF), rowb(blk, HALF))
    else:
        out_shape = jax.ShapeDtypeStruct((n, 128), F32)
        out_specs = rowb(blk, 128)
    return pl.pallas_call(kb, grid=(n // blk,), in_specs=in_specs,
                          out_specs=out_specs, out_shape=out_shape)(*args)


def _prep_edges(src, dst, junk, e_pad):
    e = src.shape[0]
    src = jnp.concatenate([src, jnp.zeros((e_pad - e,), jnp.int32)])
    dst = jnp.concatenate([dst, jnp.full((e_pad - e,), junk, jnp.int32)])
    return src.reshape(-1, 128), dst.reshape(-1, 128)


def kernel(x_item, x_user, ii_src, ii_dst, iu0_src, iu0_dst, iu1_src, iu1_dst,
           W_self1, W_neigh1, b1, W_self2, W_neigh2, b2,
           W_self3, W_neigh3, b3, W_lin, b_lin):
    zeros = jnp.zeros((128, HALF), F32)
    zeros8 = jnp.zeros((128, CW), F32)
    ones = jnp.ones((128, CW), F32)
    xiL = x_item[:, :HALF]
    xiR = x_item[:, HALF:]

    s1, d1 = _prep_edges(ii_src, ii_dst, 20000, 606208)
    sum1L, sum1R, cnt1 = _make_agg(20480, 20480, 606208 // 128)(
        xiL, xiR, s1, d1, zeros, zeros8, ones)
    itemL, itemR = _dense_call(x_item[:20480], sum1L, sum1R, cnt1,
                               W_self1, W_neigh1[:HALF], W_neigh1[HALF:],
                               b1, 'split')

    s2, d2 = _prep_edges(iu0_src, iu0_dst, 10000, 327680)
    s3, d3 = _prep_edges(iu1_src, iu1_dst + 10240, 15240, 163840)
    s23 = jnp.concatenate([s2.reshape(NS, 160, 128),
                           s3.reshape(NS, 80, 128)], axis=1).reshape(-1, 128)
    d23 = jnp.concatenate([d2.reshape(NS, 160, 128),
                           d3.reshape(NS, 80, 128)], axis=1).reshape(-1, 128)
    sAL, sAR, cntA = _make_agg23()(
        xiL, xiR, itemL, itemR, s23, d23, zeros, zeros8, ones)
    out = _dense23_call(x_user[:5120], sAL[:5120], sAR[:5120], cntA[:5120],
                        sAL[10240:], sAR[10240:], cntA[10240:],
                        W_self2, W_neigh2[:HALF], W_neigh2[HALF:], b2,
                        W_self3, W_neigh3[:HALF], W_neigh3[HALF:], b3,
                        W_lin, b_lin)
    return out[:5000]
